# baseline (speedup 1.0000x reference)
import functools

import jax
import jax.numpy as jnp
from jax import lax
from jax.experimental import pallas as pl
from jax.experimental.pallas import tpu as pltpu
from jax.experimental.pallas import tpu_sc as plsc

_NC = 2
_NS = 16
_NW = _NC * _NS
_L = 16

_TC_COLS = 1024
_TC_ROWS = 256


@functools.lru_cache(maxsize=None)
def _build_sc(n, n_edges, n_bins):
    per_w = n // _NW
    chunk = min(16384, per_w)
    n_chunks = per_w // chunk
    iters = chunk // _L

    mesh = plsc.VectorSubcoreMesh(core_axis_name="c", subcore_axis_name="s")

    @functools.partial(
        pl.kernel,
        out_type=jax.ShapeDtypeStruct((_NW, _L), jnp.float32),
        mesh=mesh,
        compiler_params=pltpu.CompilerParams(needs_layout_passes=False),
        scratch_types=[
            pltpu.VMEM((2, chunk), jnp.float32),
            pltpu.VMEM((2, chunk), jnp.float32),
            pltpu.VMEM((_L,), jnp.float32),
            pltpu.VMEM((_L,), jnp.float32),
            pltpu.VMEM((_L,), jnp.float32),
            pltpu.SemaphoreType.DMA,
            pltpu.SemaphoreType.DMA,
            pltpu.SemaphoreType.DMA,
            pltpu.SemaphoreType.DMA,
        ],
    )
    def sc_loss(pred_hbm, tgt_hbm, edges_hbm, wts_hbm, out_hbm,
                pbuf, tbuf, ebuf, wbuf, accv, sp0, sp1, st0, st1):
        sems_p = (sp0, sp1)
        sems_t = (st0, st1)
        wid = lax.axis_index("s") * _NC + lax.axis_index("c")
        base = wid * per_w

        pltpu.sync_copy(edges_hbm, ebuf.at[pl.ds(0, n_edges)])
        pltpu.sync_copy(wts_hbm, wbuf.at[pl.ds(0, n_bins)])

        zero = jnp.zeros((_L,), jnp.float32)
        nbins_f = jnp.full((_L,), float(n_bins), jnp.float32)
        half_f = jnp.full((_L,), 0.5, jnp.float32)
        one_i = jnp.full((_L,), 1, jnp.int32)
        zero_i = jnp.zeros((_L,), jnp.int32)
        max_i = jnp.full((_L,), n_bins - 1, jnp.int32)

        lanes = lax.iota(jnp.int32, _L)
        wpad = plsc.load_gather(
            wbuf, [jnp.minimum(jnp.maximum(lanes - one_i, zero_i), max_i)])
        wbuf[...] = wpad

        def start(g):
            b = g % 2
            off = pl.multiple_of(base + g * chunk, 8)
            dp = pltpu.async_copy(pred_hbm.at[pl.ds(off, chunk)], pbuf.at[b], sems_p[b])
            dt = pltpu.async_copy(tgt_hbm.at[pl.ds(off, chunk)], tbuf.at[b], sems_t[b])
            return dp, dt

        def chunk_sum(b, acc0):
            @plsc.parallel_loop(0, iters, unroll=4, carry=acc0)
            def loop(i, acc):
                off = pl.ds(pl.multiple_of(i * _L, _L), _L)
                p16 = pbuf[b, off]
                t16 = tbuf[b, off]
                d = p16 - t16
                sq = d * d
                g = (t16 * nbins_f + half_f).astype(jnp.int32)
                elo = plsc.load_gather(ebuf, [g])
                j = g + jnp.where(t16 > elo, one_i, zero_i)
                w = plsc.load_gather(wbuf, [j])
                return acc + sq * w
            return loop

        desc = start(0)
        acc = zero
        for g in range(n_chunks):
            nxt = start(g + 1) if g + 1 < n_chunks else None
            desc[0].wait()
            desc[1].wait()
            acc = chunk_sum(g % 2, acc)
            desc = nxt

        accv[...] = acc
        pltpu.sync_copy(accv, out_hbm.at[wid])

    return sc_loss


@functools.lru_cache(maxsize=None)
def _build_tc(rows_total, row0, rows_tc, n_edges, n_bins):
    grid = rows_tc // _TC_ROWS

    def body(e_ref, w_ref, p_ref, t_ref, o_ref):
        i = pl.program_id(0)
        p = p_ref[...]
        t = t_ref[...]
        sq = (p - t) * (p - t)
        w = jnp.full(p.shape, w_ref[0], jnp.float32)
        for k in range(1, n_bins):
            w += jnp.where(t > e_ref[k], w_ref[k] - w_ref[k - 1],
                           jnp.float32(0.0))

        @pl.when(i == 0)
        def _():
            o_ref[0, 0] = jnp.float32(0.0)

        o_ref[0, 0] += jnp.sum(sq * w)

    return pl.pallas_call(
        body,
        grid=(grid,),
        in_specs=[
            pl.BlockSpec(memory_space=pltpu.SMEM),
            pl.BlockSpec(memory_space=pltpu.SMEM),
            pl.BlockSpec((_TC_ROWS, _TC_COLS), lambda i: (row0 + i, 0)),
            pl.BlockSpec((_TC_ROWS, _TC_COLS), lambda i: (row0 + i, 0)),
        ],
        out_specs=pl.BlockSpec(memory_space=pltpu.SMEM),
        out_shape=jax.ShapeDtypeStruct((1, 1), jnp.float32),
    )


@jax.jit
def kernel(predictions, targets, bin_edges, bin_weights):
    predictions = jnp.squeeze(predictions)
    targets = jnp.squeeze(targets)
    n = predictions.shape[0]
    n_edges = bin_edges.shape[0]
    n_bins = bin_weights.shape[0]

    sc_quantum = _NW * 16384
    if n % (2 * sc_quantum) == 0 and (n // 2) % (_TC_ROWS * _TC_COLS) == 0:
        n_sc = n // 2
        rows_total = n // _TC_COLS
        row0 = (n_sc // _TC_COLS) // _TC_ROWS
        rows_tc = rows_total - n_sc // _TC_COLS
        sc_loss = _build_sc(n_sc, n_edges, n_bins)
        tc_loss = _build_tc(rows_total, row0, rows_tc, n_edges, n_bins)
        partials = sc_loss(predictions, targets, bin_edges, bin_weights)
        p2 = predictions.reshape(rows_total, _TC_COLS)
        t2 = targets.reshape(rows_total, _TC_COLS)
        tc_part = tc_loss(bin_edges, bin_weights, p2, t2)
        total = jnp.sum(partials) + tc_part[0, 0]
    else:
        sc_loss = _build_sc(n, n_edges, n_bins)
        partials = sc_loss(predictions, targets, bin_edges, bin_weights)
        total = jnp.sum(partials)
    return total / jnp.float32(n)


# speedup vs baseline: 1.5221x; 1.5221x over previous
import functools

import jax
import jax.numpy as jnp
from jax import lax
from jax.experimental import pallas as pl
from jax.experimental.pallas import tpu as pltpu
from jax.experimental.pallas import tpu_sc as plsc

_NC = 2
_NS = 16
_NW = _NC * _NS
_L = 16

_TC_COLS = 1024
_TC_ROWS = 256


@functools.lru_cache(maxsize=None)
def _build_sc(n, n_edges, n_bins):
    per_w = n // _NW
    chunk = min(16384, per_w)
    n_chunks = per_w // chunk
    iters = chunk // _L

    mesh = plsc.VectorSubcoreMesh(core_axis_name="c", subcore_axis_name="s")

    @functools.partial(
        pl.kernel,
        out_type=jax.ShapeDtypeStruct((_NW, _L), jnp.float32),
        mesh=mesh,
        compiler_params=pltpu.CompilerParams(needs_layout_passes=False),
        scratch_types=[
            pltpu.VMEM((2, chunk), jnp.float32),
            pltpu.VMEM((2, chunk), jnp.float32),
            pltpu.VMEM((_L,), jnp.float32),
            pltpu.VMEM((_L,), jnp.float32),
            pltpu.VMEM((_L,), jnp.float32),
            pltpu.SemaphoreType.DMA,
            pltpu.SemaphoreType.DMA,
            pltpu.SemaphoreType.DMA,
            pltpu.SemaphoreType.DMA,
        ],
    )
    def sc_loss(pred_hbm, tgt_hbm, edges_hbm, wts_hbm, out_hbm,
                pbuf, tbuf, ebuf, wbuf, accv, sp0, sp1, st0, st1):
        sems_p = (sp0, sp1)
        sems_t = (st0, st1)
        wid = lax.axis_index("s") * _NC + lax.axis_index("c")
        base = wid * per_w

        pltpu.sync_copy(edges_hbm, ebuf.at[pl.ds(0, n_edges)])
        pltpu.sync_copy(wts_hbm, wbuf.at[pl.ds(0, n_bins)])

        zero = jnp.zeros((_L,), jnp.float32)
        nbins_f = jnp.full((_L,), float(n_bins), jnp.float32)
        half_f = jnp.full((_L,), 0.5, jnp.float32)
        one_i = jnp.full((_L,), 1, jnp.int32)
        zero_i = jnp.zeros((_L,), jnp.int32)
        max_i = jnp.full((_L,), n_bins - 1, jnp.int32)

        lanes = lax.iota(jnp.int32, _L)
        wpad = plsc.load_gather(
            wbuf, [jnp.minimum(jnp.maximum(lanes - one_i, zero_i), max_i)])
        wbuf[...] = wpad

        def start(g):
            b = g % 2
            off = pl.multiple_of(base + g * chunk, 8)
            dp = pltpu.async_copy(pred_hbm.at[pl.ds(off, chunk)], pbuf.at[b], sems_p[b])
            dt = pltpu.async_copy(tgt_hbm.at[pl.ds(off, chunk)], tbuf.at[b], sems_t[b])
            return dp, dt

        def chunk_sum(b, acc0):
            @plsc.parallel_loop(0, iters, unroll=4, carry=acc0)
            def loop(i, acc):
                off = pl.ds(pl.multiple_of(i * _L, _L), _L)
                p16 = pbuf[b, off]
                t16 = tbuf[b, off]
                d = p16 - t16
                sq = d * d
                g = (t16 * nbins_f + half_f).astype(jnp.int32)
                elo = plsc.load_gather(ebuf, [g])
                j = g + jnp.where(t16 > elo, one_i, zero_i)
                w = plsc.load_gather(wbuf, [j])
                return acc + sq * w
            return loop

        desc = start(0)
        acc = zero
        for g in range(n_chunks):
            nxt = start(g + 1) if g + 1 < n_chunks else None
            desc[0].wait()
            desc[1].wait()
            acc = chunk_sum(g % 2, acc)
            desc = nxt

        accv[...] = acc
        pltpu.sync_copy(accv, out_hbm.at[wid])

    return sc_loss


_TC_BLK = _TC_ROWS * _TC_COLS


@functools.lru_cache(maxsize=None)
def _build_tc(blk0, n_blocks, n_edges, n_bins):
    def body(e_ref, w_ref, p_ref, t_ref, o_ref):
        i = pl.program_id(0)
        p = p_ref[...]
        t = t_ref[...]
        sq = (p - t) * (p - t)
        w = jnp.full(p.shape, w_ref[0], jnp.float32)
        for k in range(1, n_bins):
            w += jnp.where(t > e_ref[k], w_ref[k] - w_ref[k - 1],
                           jnp.float32(0.0))

        @pl.when(i == 0)
        def _():
            o_ref[0, 0] = jnp.float32(0.0)

        o_ref[0, 0] += jnp.sum(sq * w)

    return pl.pallas_call(
        body,
        grid=(n_blocks,),
        in_specs=[
            pl.BlockSpec(memory_space=pltpu.SMEM),
            pl.BlockSpec(memory_space=pltpu.SMEM),
            pl.BlockSpec((_TC_BLK,), lambda i: (blk0 + i,)),
            pl.BlockSpec((_TC_BLK,), lambda i: (blk0 + i,)),
        ],
        out_specs=pl.BlockSpec(memory_space=pltpu.SMEM),
        out_shape=jax.ShapeDtypeStruct((1, 1), jnp.float32),
    )


@jax.jit
def kernel(predictions, targets, bin_edges, bin_weights):
    predictions = jnp.squeeze(predictions)
    targets = jnp.squeeze(targets)
    n = predictions.shape[0]
    n_edges = bin_edges.shape[0]
    n_bins = bin_weights.shape[0]

    sc_quantum = _NW * 16384
    if n % (2 * sc_quantum) == 0 and (n // 2) % _TC_BLK == 0:
        n_sc = n // 2
        blk0 = n_sc // _TC_BLK
        n_blocks = (n - n_sc) // _TC_BLK
        sc_loss = _build_sc(n_sc, n_edges, n_bins)
        tc_loss = _build_tc(blk0, n_blocks, n_edges, n_bins)
        partials = sc_loss(predictions, targets, bin_edges, bin_weights)
        tc_part = tc_loss(bin_edges, bin_weights, predictions, targets)
        total = jnp.sum(partials) + tc_part[0, 0]
    else:
        sc_loss = _build_sc(n, n_edges, n_bins)
        partials = sc_loss(predictions, targets, bin_edges, bin_weights)
        total = jnp.sum(partials)
    return total / jnp.float32(n)


# speedup vs baseline: 2.0836x; 1.3688x over previous
import functools

import jax
import jax.numpy as jnp
from jax import lax
from jax.experimental import pallas as pl
from jax.experimental.pallas import tpu as pltpu
from jax.experimental.pallas import tpu_sc as plsc

_NC = 2
_NS = 16
_NW = _NC * _NS
_L = 16

_TC_COLS = 1024
_TC_ROWS = 256


@functools.lru_cache(maxsize=None)
def _build_sc(n, n_edges, n_bins):
    per_w = n // _NW
    chunk = min(16384, per_w)
    n_chunks = per_w // chunk
    iters = chunk // _L

    mesh = plsc.VectorSubcoreMesh(core_axis_name="c", subcore_axis_name="s")

    @functools.partial(
        pl.kernel,
        out_type=jax.ShapeDtypeStruct((_NW, _L), jnp.float32),
        mesh=mesh,
        compiler_params=pltpu.CompilerParams(needs_layout_passes=False),
        scratch_types=[
            pltpu.VMEM((2, chunk), jnp.float32),
            pltpu.VMEM((2, chunk), jnp.float32),
            pltpu.VMEM((_L,), jnp.float32),
            pltpu.VMEM((_L,), jnp.float32),
            pltpu.VMEM((_L,), jnp.float32),
            pltpu.SemaphoreType.DMA,
            pltpu.SemaphoreType.DMA,
            pltpu.SemaphoreType.DMA,
            pltpu.SemaphoreType.DMA,
        ],
    )
    def sc_loss(pred_hbm, tgt_hbm, edges_hbm, wts_hbm, out_hbm,
                pbuf, tbuf, ebuf, wbuf, accv, sp0, sp1, st0, st1):
        sems_p = (sp0, sp1)
        sems_t = (st0, st1)
        wid = lax.axis_index("s") * _NC + lax.axis_index("c")
        base = wid * per_w

        pltpu.sync_copy(edges_hbm, ebuf.at[pl.ds(0, n_edges)])
        pltpu.sync_copy(wts_hbm, wbuf.at[pl.ds(0, n_bins)])

        zero = jnp.zeros((_L,), jnp.float32)
        nbins_f = jnp.full((_L,), float(n_bins), jnp.float32)
        half_f = jnp.full((_L,), 0.5, jnp.float32)
        one_i = jnp.full((_L,), 1, jnp.int32)
        zero_i = jnp.zeros((_L,), jnp.int32)
        max_i = jnp.full((_L,), n_bins - 1, jnp.int32)

        lanes = lax.iota(jnp.int32, _L)
        wpad = plsc.load_gather(
            wbuf, [jnp.minimum(jnp.maximum(lanes - one_i, zero_i), max_i)])
        wbuf[...] = wpad

        def start(g):
            b = g % 2
            off = pl.multiple_of(base + g * chunk, 8)
            dp = pltpu.async_copy(pred_hbm.at[pl.ds(off, chunk)], pbuf.at[b], sems_p[b])
            dt = pltpu.async_copy(tgt_hbm.at[pl.ds(off, chunk)], tbuf.at[b], sems_t[b])
            return dp, dt

        def chunk_sum(b, acc0):
            @plsc.parallel_loop(0, iters, unroll=4, carry=acc0)
            def loop(i, acc):
                off = pl.ds(pl.multiple_of(i * _L, _L), _L)
                p16 = pbuf[b, off]
                t16 = tbuf[b, off]
                d = p16 - t16
                sq = d * d
                g = (t16 * nbins_f + half_f).astype(jnp.int32)
                elo = plsc.load_gather(ebuf, [g])
                j = g + jnp.where(t16 > elo, one_i, zero_i)
                w = plsc.load_gather(wbuf, [j])
                return acc + sq * w
            return loop

        desc = start(0)
        acc = zero
        for g in range(n_chunks):
            nxt = start(g + 1) if g + 1 < n_chunks else None
            desc[0].wait()
            desc[1].wait()
            acc = chunk_sum(g % 2, acc)
            desc = nxt

        accv[...] = acc
        pltpu.sync_copy(accv, out_hbm.at[wid])

    return sc_loss


_TC_BLK_ROWS = 2048
_TC_BLK = _TC_BLK_ROWS * 128


@functools.lru_cache(maxsize=None)
def _build_tc(blk0, n_blocks, n_edges, n_bins):
    def body(e_ref, w_ref, p_ref, t_ref, o_ref):
        i = pl.program_id(0)
        p = p_ref[...]
        t = t_ref[...]
        sq = (p - t) * (p - t)
        w = jnp.full(p.shape, w_ref[0], jnp.float32)
        for k in range(1, n_bins):
            w += jnp.where(t > e_ref[k], w_ref[k] - w_ref[k - 1],
                           jnp.float32(0.0))

        @pl.when(i == 0)
        def _():
            o_ref[0, 0] = jnp.float32(0.0)

        o_ref[0, 0] += jnp.sum(sq * w)

    return pl.pallas_call(
        body,
        grid=(n_blocks,),
        in_specs=[
            pl.BlockSpec(memory_space=pltpu.SMEM),
            pl.BlockSpec(memory_space=pltpu.SMEM),
            pl.BlockSpec((_TC_BLK_ROWS, 128), lambda i: (blk0 + i, 0)),
            pl.BlockSpec((_TC_BLK_ROWS, 128), lambda i: (blk0 + i, 0)),
        ],
        out_specs=pl.BlockSpec(memory_space=pltpu.SMEM),
        out_shape=jax.ShapeDtypeStruct((1, 1), jnp.float32),
    )


@jax.jit
def kernel(predictions, targets, bin_edges, bin_weights):
    predictions = jnp.squeeze(predictions)
    targets = jnp.squeeze(targets)
    n = predictions.shape[0]
    n_edges = bin_edges.shape[0]
    n_bins = bin_weights.shape[0]

    sc_quantum = _NW * 16384
    if n % (2 * sc_quantum) == 0 and (n // 2) % _TC_BLK == 0:
        n_sc = n // 2
        blk0 = n_sc // _TC_BLK
        n_blocks = (n - n_sc) // _TC_BLK
        sc_loss = _build_sc(n_sc, n_edges, n_bins)
        tc_loss = _build_tc(blk0, n_blocks, n_edges, n_bins)
        partials = sc_loss(predictions, targets, bin_edges, bin_weights)
        p2 = predictions.reshape(n // 128, 128)
        t2 = targets.reshape(n // 128, 128)
        tc_part = tc_loss(bin_edges, bin_weights, p2, t2)
        total = jnp.sum(partials) + tc_part[0, 0]
    else:
        sc_loss = _build_sc(n, n_edges, n_bins)
        partials = sc_loss(predictions, targets, bin_edges, bin_weights)
        total = jnp.sum(partials)
    return total / jnp.float32(n)


# speedup vs baseline: 2.2252x; 1.0680x over previous
import functools

import jax
import jax.numpy as jnp
from jax import lax
from jax.experimental import pallas as pl
from jax.experimental.pallas import tpu as pltpu
from jax.experimental.pallas import tpu_sc as plsc

_NC = 2
_NS = 16
_NW = _NC * _NS
_L = 16

_TC_COLS = 1024
_TC_ROWS = 256


@functools.lru_cache(maxsize=None)
def _build_sc(n, n_edges, n_bins):
    per_w = n // _NW
    chunk = min(16384, per_w)
    n_chunks = per_w // chunk
    iters = chunk // _L

    mesh = plsc.VectorSubcoreMesh(core_axis_name="c", subcore_axis_name="s")

    @functools.partial(
        pl.kernel,
        out_type=jax.ShapeDtypeStruct((_NW, _L), jnp.float32),
        mesh=mesh,
        compiler_params=pltpu.CompilerParams(needs_layout_passes=False),
        scratch_types=[
            pltpu.VMEM((2, chunk), jnp.float32),
            pltpu.VMEM((2, chunk), jnp.float32),
            pltpu.VMEM((_L,), jnp.float32),
            pltpu.VMEM((_L,), jnp.float32),
            pltpu.VMEM((_L,), jnp.float32),
            pltpu.SemaphoreType.DMA,
            pltpu.SemaphoreType.DMA,
            pltpu.SemaphoreType.DMA,
            pltpu.SemaphoreType.DMA,
        ],
    )
    def sc_loss(pred_hbm, tgt_hbm, edges_hbm, wts_hbm, out_hbm,
                pbuf, tbuf, ebuf, wbuf, accv, sp0, sp1, st0, st1):
        sems_p = (sp0, sp1)
        sems_t = (st0, st1)
        wid = lax.axis_index("s") * _NC + lax.axis_index("c")
        base = wid * per_w

        pltpu.sync_copy(edges_hbm, ebuf.at[pl.ds(0, n_edges)])
        pltpu.sync_copy(wts_hbm, wbuf.at[pl.ds(0, n_bins)])

        zero = jnp.zeros((_L,), jnp.float32)
        nbins_f = jnp.full((_L,), float(n_bins), jnp.float32)
        half_f = jnp.full((_L,), 0.5, jnp.float32)
        one_i = jnp.full((_L,), 1, jnp.int32)
        zero_i = jnp.zeros((_L,), jnp.int32)
        max_i = jnp.full((_L,), n_bins - 1, jnp.int32)

        lanes = lax.iota(jnp.int32, _L)
        wpad = plsc.load_gather(
            wbuf, [jnp.minimum(jnp.maximum(lanes - one_i, zero_i), max_i)])
        wbuf[...] = wpad

        def start(g):
            b = g % 2
            off = pl.multiple_of(base + g * chunk, 8)
            dp = pltpu.async_copy(pred_hbm.at[pl.ds(off, chunk)], pbuf.at[b], sems_p[b])
            dt = pltpu.async_copy(tgt_hbm.at[pl.ds(off, chunk)], tbuf.at[b], sems_t[b])
            return dp, dt

        def chunk_sum(b, acc0):
            @plsc.parallel_loop(0, iters, unroll=4, carry=acc0)
            def loop(i, acc):
                off = pl.ds(pl.multiple_of(i * _L, _L), _L)
                p16 = pbuf[b, off]
                t16 = tbuf[b, off]
                d = p16 - t16
                sq = d * d
                g = (t16 * nbins_f + half_f).astype(jnp.int32)
                elo = plsc.load_gather(ebuf, [g])
                j = g + jnp.where(t16 > elo, one_i, zero_i)
                w = plsc.load_gather(wbuf, [j])
                return acc + sq * w
            return loop

        desc = start(0)
        acc = zero
        for g in range(n_chunks):
            nxt = start(g + 1) if g + 1 < n_chunks else None
            desc[0].wait()
            desc[1].wait()
            acc = chunk_sum(g % 2, acc)
            desc = nxt

        accv[...] = acc
        pltpu.sync_copy(accv, out_hbm.at[wid])

    return sc_loss


_TC_BLK_ROWS = 4096
_TC_BLK = _TC_BLK_ROWS * 128


@functools.lru_cache(maxsize=None)
def _build_tc(blk0, n_blocks, n_edges, n_bins):
    def body(e_ref, w_ref, p_ref, t_ref, o_ref):
        i = pl.program_id(0)
        p = p_ref[...]
        t = t_ref[...]
        sq = (p - t) * (p - t)
        w = jnp.full(p.shape, w_ref[0], jnp.float32)
        for k in range(1, n_bins):
            w += jnp.where(t > e_ref[k], w_ref[k] - w_ref[k - 1],
                           jnp.float32(0.0))

        @pl.when(i == 0)
        def _():
            o_ref[0, 0] = jnp.float32(0.0)

        o_ref[0, 0] += jnp.sum(sq * w)

    return pl.pallas_call(
        body,
        grid=(n_blocks,),
        in_specs=[
            pl.BlockSpec(memory_space=pltpu.SMEM),
            pl.BlockSpec(memory_space=pltpu.SMEM),
            pl.BlockSpec((_TC_BLK_ROWS, 128), lambda i: (blk0 + i, 0)),
            pl.BlockSpec((_TC_BLK_ROWS, 128), lambda i: (blk0 + i, 0)),
        ],
        out_specs=pl.BlockSpec(memory_space=pltpu.SMEM),
        out_shape=jax.ShapeDtypeStruct((1, 1), jnp.float32),
    )


@jax.jit
def kernel(predictions, targets, bin_edges, bin_weights):
    predictions = jnp.squeeze(predictions)
    targets = jnp.squeeze(targets)
    n = predictions.shape[0]
    n_edges = bin_edges.shape[0]
    n_bins = bin_weights.shape[0]

    sc_quantum = _NW * 16384
    if n % (16 * sc_quantum) == 0 and (n - 7 * (n // 16)) % _TC_BLK == 0:
        n_sc = 7 * (n // 16)
        blk0 = n_sc // _TC_BLK
        n_blocks = (n - n_sc) // _TC_BLK
        sc_loss = _build_sc(n_sc, n_edges, n_bins)
        tc_loss = _build_tc(blk0, n_blocks, n_edges, n_bins)
        partials = sc_loss(predictions, targets, bin_edges, bin_weights)
        p2 = predictions.reshape(n // 128, 128)
        t2 = targets.reshape(n // 128, 128)
        tc_part = tc_loss(bin_edges, bin_weights, p2, t2)
        total = jnp.sum(partials) + tc_part[0, 0]
    else:
        sc_loss = _build_sc(n, n_edges, n_bins)
        partials = sc_loss(predictions, targets, bin_edges, bin_weights)
        total = jnp.sum(partials)
    return total / jnp.float32(n)
